# Initial kernel scaffold; baseline (speedup 1.0000x reference)
import functools

import jax
import jax.numpy as jnp
from jax import lax
from jax.experimental import pallas as pl
from jax.experimental.pallas import tpu as pltpu
from jax.experimental.pallas import tpu_sc as plsc

N = 10000
E = 320000
D = 128

NC = 2
NS = 16
NW = NC * NS

EDGES_PER_WORKER = E // NW
BLK = 80
NBLK = EDGES_PER_WORKER // BLK
ROWS_PER_TILE = N // NS
RCHUNK = 125
NRCHUNK = ROWS_PER_TILE // RCHUNK


def _sc_aggregate(ndata, src, dst, zrows, zdeg, ones_blk):
    mesh = plsc.VectorSubcoreMesh(core_axis_name="c", subcore_axis_name="s")

    @functools.partial(
        pl.kernel,
        mesh=mesh,
        out_type=(
            jax.ShapeDtypeStruct((NC, N, D), jnp.float32),
            jax.ShapeDtypeStruct((NC, N, 1), jnp.float32),
        ),
        scratch_types=[
            pltpu.VMEM((BLK,), jnp.int32),
            pltpu.VMEM((BLK,), jnp.int32),
            pltpu.VMEM((BLK, D), jnp.float32),
            pltpu.VMEM((RCHUNK, D), jnp.float32),
            pltpu.VMEM((ROWS_PER_TILE, 1), jnp.float32),
            pltpu.VMEM((BLK, 1), jnp.float32),
            pltpu.VMEM_SHARED((N, D), jnp.float32),
            pltpu.VMEM_SHARED((N, 1), jnp.float32),
            pltpu.SemaphoreType.DMA,
        ],
    )
    def k(ndata_hbm, src_hbm, dst_hbm, zrows_hbm, zdeg_hbm, ones_hbm,
          acc_out, deg_out, sidx, didx, rows, stage, dstage, ones_v,
          acc, dacc, sem):
        c = lax.axis_index("c")
        s = lax.axis_index("s")
        tile_base = s * ROWS_PER_TILE

        pltpu.sync_copy(zrows_hbm, stage)
        for kk in range(NRCHUNK):
            pltpu.sync_copy(stage, acc.at[pl.ds(tile_base + kk * RCHUNK, RCHUNK)])
        pltpu.sync_copy(zdeg_hbm, dstage)
        pltpu.sync_copy(dstage, dacc.at[pl.ds(tile_base, ROWS_PER_TILE)])
        pltpu.sync_copy(ones_hbm, ones_v)
        plsc.subcore_barrier()

        chunk_base = (c * NS + s) * EDGES_PER_WORKER

        def body(i, carry):
            base = chunk_base + i * BLK
            pltpu.sync_copy(src_hbm.at[pl.ds(base, BLK)], sidx)
            pltpu.sync_copy(dst_hbm.at[pl.ds(base, BLK)], didx)
            pltpu.async_copy(ndata_hbm.at[sidx], rows, sem).wait()
            pltpu.sync_copy(rows, acc.at[didx], add=True)
            pltpu.sync_copy(ones_v, dacc.at[didx], add=True)
            return carry

        lax.fori_loop(0, NBLK, body, 0)
        plsc.subcore_barrier()

        for kk in range(NRCHUNK):
            r0 = tile_base + kk * RCHUNK
            pltpu.sync_copy(acc.at[pl.ds(r0, RCHUNK)], stage)
            pltpu.sync_copy(stage, acc_out.at[c, pl.ds(r0, RCHUNK)])
        pltpu.sync_copy(dacc.at[pl.ds(tile_base, ROWS_PER_TILE)], dstage)
        pltpu.sync_copy(dstage, deg_out.at[c, pl.ds(tile_base, ROWS_PER_TILE)])

    return k(ndata, src, dst, zrows, zdeg, ones_blk)


ROW_BLK = 400


def _tc_finish_body(nd_ref, p0_ref, p1_ref, d0_ref, d1_ref, wt_ref, b_ref,
                    o_ref):
    deg = d0_ref[...] + d1_ref[...]
    agg = (p0_ref[...] + p1_ref[...]) / jnp.maximum(deg, 1.0)
    rst = nd_ref[...] + agg
    o_ref[...] = (
        jnp.dot(rst, wt_ref[...], preferred_element_type=jnp.float32)
        + b_ref[...]
    )


def _tc_finish(ndata, p0, p1, d0, d1, wt, b2):
    grid = (N // ROW_BLK,)
    return pl.pallas_call(
        _tc_finish_body,
        grid=grid,
        in_specs=[
            pl.BlockSpec((ROW_BLK, D), lambda i: (i, 0)),
            pl.BlockSpec((ROW_BLK, D), lambda i: (i, 0)),
            pl.BlockSpec((ROW_BLK, D), lambda i: (i, 0)),
            pl.BlockSpec((ROW_BLK, 1), lambda i: (i, 0)),
            pl.BlockSpec((ROW_BLK, 1), lambda i: (i, 0)),
            pl.BlockSpec((D, D), lambda i: (0, 0)),
            pl.BlockSpec((1, D), lambda i: (0, 0)),
        ],
        out_specs=pl.BlockSpec((ROW_BLK, D), lambda i: (i, 0)),
        out_shape=jax.ShapeDtypeStruct((N, D), jnp.float32),
    )(ndata, p0, p1, d0, d1, wt, b2)


@jax.jit
def kernel(ndata, edge_index, W, b):
    src = edge_index[0]
    dst = edge_index[1]
    zrows = jnp.zeros((RCHUNK, D), jnp.float32)
    zdeg = jnp.zeros((ROWS_PER_TILE, 1), jnp.float32)
    ones_blk = jnp.ones((BLK, 1), jnp.float32)

    acc, deg = _sc_aggregate(ndata, src, dst, zrows, zdeg, ones_blk)

    wt = W.T
    b2 = b.reshape(1, D)
    return _tc_finish(ndata, acc[0], acc[1], deg[0], deg[1], wt, b2)


# speedup vs baseline: 5.8243x; 5.8243x over previous
import functools

import jax
import jax.numpy as jnp
from jax import lax
from jax.experimental import pallas as pl
from jax.experimental.pallas import tpu as pltpu
from jax.experimental.pallas import tpu_sc as plsc

N = 10000
E = 320000
D = 128

NC = 2
NS = 16
NW = NC * NS

EDGES_PER_WORKER = E // NW
BLK = 80
NBLK = EDGES_PER_WORKER // BLK
RCHUNK = 80
NRCHUNK = N // RCHUNK
ROUNDS = (NRCHUNK + NS - 1) // NS


def _sc_aggregate(ndata, src, dst, zrows, zdeg, ones_blk):
    mesh = plsc.VectorSubcoreMesh(core_axis_name="c", subcore_axis_name="s")

    @functools.partial(
        pl.kernel,
        mesh=mesh,
        out_type=(
            jax.ShapeDtypeStruct((NC, N, D), jnp.float32),
            jax.ShapeDtypeStruct((N,), jnp.float32),
            jax.ShapeDtypeStruct((N,), jnp.float32),
        ),
        scratch_types=[
            pltpu.VMEM((BLK,), jnp.int32),
            pltpu.VMEM((BLK,), jnp.int32),
            pltpu.VMEM((BLK, D), jnp.float32),
            pltpu.VMEM((RCHUNK,), jnp.float32),
            pltpu.VMEM((BLK,), jnp.float32),
            pltpu.VMEM_SHARED((N, D), jnp.float32),
            pltpu.VMEM_SHARED((N,), jnp.float32),
            pltpu.SemaphoreType.DMA,
        ],
    )
    def k(ndata_hbm, src_hbm, dst_hbm, zrows_hbm, zdeg_hbm, ones_hbm,
          acc_out, deg0_out, deg1_out, sidx, didx, rows, dstage,
          ones_v, acc, dacc, sem):
        c = lax.axis_index("c")
        s = lax.axis_index("s")

        pltpu.sync_copy(zrows_hbm, rows)
        pltpu.sync_copy(zdeg_hbm, dstage)
        pltpu.sync_copy(ones_hbm, ones_v)
        for kk in range(ROUNDS):
            cid = s + NS * kk

            @pl.when(cid < NRCHUNK)
            def _():
                r0 = cid * RCHUNK
                pltpu.sync_copy(rows, acc.at[pl.ds(r0, RCHUNK)])
                pltpu.sync_copy(dstage, dacc.at[pl.ds(r0, RCHUNK)])

        plsc.subcore_barrier()

        chunk_base = (c * NS + s) * EDGES_PER_WORKER

        def body(i, carry):
            base = chunk_base + i * BLK
            pltpu.sync_copy(src_hbm.at[pl.ds(base, BLK)], sidx)
            pltpu.sync_copy(dst_hbm.at[pl.ds(base, BLK)], didx)
            pltpu.async_copy(ndata_hbm.at[sidx], rows, sem).wait()
            pltpu.sync_copy(rows, acc.at[didx], add=True)
            pltpu.sync_copy(ones_v, dacc.at[didx], add=True)
            return carry

        lax.fori_loop(0, NBLK, body, 0)
        plsc.subcore_barrier()

        for kk in range(ROUNDS):
            cid = s + NS * kk

            @pl.when(cid < NRCHUNK)
            def _():
                r0 = cid * RCHUNK
                pltpu.sync_copy(acc.at[pl.ds(r0, RCHUNK)], rows)
                pltpu.sync_copy(rows, acc_out.at[c, pl.ds(r0, RCHUNK)])
                pltpu.sync_copy(dacc.at[pl.ds(r0, RCHUNK)], dstage)

                @pl.when(c == 0)
                def _():
                    pltpu.sync_copy(dstage, deg0_out.at[pl.ds(r0, RCHUNK)])

                @pl.when(c == 1)
                def _():
                    pltpu.sync_copy(dstage, deg1_out.at[pl.ds(r0, RCHUNK)])

    return k(ndata, src, dst, zrows, zdeg, ones_blk)


ROW_BLK = 400


def _tc_finish_body(nd_ref, p0_ref, p1_ref, d0_ref, d1_ref, wt_ref, b_ref,
                    o_ref):
    deg = d0_ref[...] + d1_ref[...]
    agg = (p0_ref[...] + p1_ref[...]) / jnp.maximum(deg, 1.0)
    rst = nd_ref[...] + agg
    o_ref[...] = (
        jnp.dot(rst, wt_ref[...], preferred_element_type=jnp.float32)
        + b_ref[...]
    )


def _tc_finish(ndata, p0, p1, d0, d1, wt, b2):
    grid = (N // ROW_BLK,)
    return pl.pallas_call(
        _tc_finish_body,
        grid=grid,
        in_specs=[
            pl.BlockSpec((ROW_BLK, D), lambda i: (i, 0)),
            pl.BlockSpec((ROW_BLK, D), lambda i: (i, 0)),
            pl.BlockSpec((ROW_BLK, D), lambda i: (i, 0)),
            pl.BlockSpec((ROW_BLK, 1), lambda i: (i, 0)),
            pl.BlockSpec((ROW_BLK, 1), lambda i: (i, 0)),
            pl.BlockSpec((D, D), lambda i: (0, 0)),
            pl.BlockSpec((1, D), lambda i: (0, 0)),
        ],
        out_specs=pl.BlockSpec((ROW_BLK, D), lambda i: (i, 0)),
        out_shape=jax.ShapeDtypeStruct((N, D), jnp.float32),
    )(ndata, p0, p1, d0, d1, wt, b2)


@jax.jit
def kernel(ndata, edge_index, W, b):
    src = edge_index[0]
    dst = edge_index[1]
    zrows = jnp.zeros((RCHUNK, D), jnp.float32)
    zdeg = jnp.zeros((RCHUNK,), jnp.float32)
    ones_blk = jnp.ones((BLK,), jnp.float32)

    acc, deg0, deg1 = _sc_aggregate(ndata, src, dst, zrows, zdeg, ones_blk)

    wt = W.T
    b2 = b.reshape(1, D)
    return _tc_finish(ndata, acc[0], acc[1], deg0.reshape(N, 1),
                      deg1.reshape(N, 1), wt, b2)


# speedup vs baseline: 11.7843x; 2.0233x over previous
import functools

import jax
import jax.numpy as jnp
from jax import lax
from jax.experimental import pallas as pl
from jax.experimental.pallas import tpu as pltpu
from jax.experimental.pallas import tpu_sc as plsc

N = 10000
E = 320000
D = 128

NC = 2
NS = 16
NW = NC * NS

EDGES_PER_WORKER = E // NW
BLK = 80
NBLK = EDGES_PER_WORKER // BLK
RCHUNK = 80
NRCHUNK = N // RCHUNK
ROUNDS = (NRCHUNK + NS - 1) // NS


def _sc_aggregate(ndata, src, dst, zrows, zdeg, ones_blk):
    mesh = plsc.VectorSubcoreMesh(core_axis_name="c", subcore_axis_name="s")

    @functools.partial(
        pl.kernel,
        mesh=mesh,
        out_type=(
            jax.ShapeDtypeStruct((NC, N, D), jnp.float32),
            jax.ShapeDtypeStruct((N,), jnp.float32),
            jax.ShapeDtypeStruct((N,), jnp.float32),
        ),
        scratch_types=[
            pltpu.VMEM((3, BLK), jnp.int32),
            pltpu.VMEM((3, BLK), jnp.int32),
            pltpu.VMEM((2, BLK, D), jnp.float32),
            pltpu.VMEM((RCHUNK,), jnp.float32),
            pltpu.VMEM((BLK,), jnp.float32),
            pltpu.VMEM_SHARED((N, D), jnp.float32),
            pltpu.VMEM_SHARED((N,), jnp.float32),
            pltpu.SemaphoreType.DMA((3,)),
            pltpu.SemaphoreType.DMA((2,)),
            pltpu.SemaphoreType.DMA,
        ],
    )
    def k(ndata_hbm, src_hbm, dst_hbm, zrows_hbm, zdeg_hbm, ones_hbm,
          acc_out, deg0_out, deg1_out, sidx, didx, rows, dstage,
          ones_v, acc, dacc, semi, semg, sems):
        c = lax.axis_index("c")
        s = lax.axis_index("s")

        pltpu.sync_copy(zrows_hbm, rows.at[0])
        pltpu.sync_copy(zdeg_hbm, dstage)
        pltpu.sync_copy(ones_hbm, ones_v)
        for kk in range(ROUNDS):
            cid = s + NS * kk

            @pl.when(cid < NRCHUNK)
            def _():
                r0 = cid * RCHUNK
                pltpu.sync_copy(rows.at[0], acc.at[pl.ds(r0, RCHUNK)])
                pltpu.sync_copy(dstage, dacc.at[pl.ds(r0, RCHUNK)])

        plsc.subcore_barrier()

        chunk_base = (c * NS + s) * EDGES_PER_WORKER

        def fire_idx(blk):
            isl = lax.rem(blk, 3)
            base = chunk_base + blk * BLK
            pltpu.async_copy(src_hbm.at[pl.ds(base, BLK)], sidx.at[isl],
                             semi.at[isl])
            pltpu.async_copy(dst_hbm.at[pl.ds(base, BLK)], didx.at[isl],
                             semi.at[isl])

        def wait_idx(blk):
            isl = lax.rem(blk, 3)
            base = chunk_base + blk * BLK
            pltpu.make_async_copy(src_hbm.at[pl.ds(base, BLK)],
                                  sidx.at[isl], semi.at[isl]).wait()
            pltpu.make_async_copy(dst_hbm.at[pl.ds(base, BLK)],
                                  didx.at[isl], semi.at[isl]).wait()

        def fire_gather(blk):
            isl = lax.rem(blk, 3)
            rsl = lax.rem(blk, 2)
            pltpu.async_copy(ndata_hbm.at[sidx.at[isl]], rows.at[rsl],
                             semg.at[rsl])

        def wait_gather(blk):
            isl = lax.rem(blk, 3)
            rsl = lax.rem(blk, 2)
            pltpu.make_async_copy(ndata_hbm.at[sidx.at[isl]], rows.at[rsl],
                                  semg.at[rsl]).wait()

        def fire_scat(blk):
            isl = lax.rem(blk, 3)
            rsl = lax.rem(blk, 2)
            pltpu.async_copy(rows.at[rsl], acc.at[didx.at[isl]], sems,
                             add=True)
            pltpu.async_copy(ones_v, dacc.at[didx.at[isl]], sems, add=True)

        def wait_scat(blk):
            isl = lax.rem(blk, 3)
            rsl = lax.rem(blk, 2)
            pltpu.make_async_copy(rows.at[rsl], acc.at[didx.at[isl]],
                                  sems).wait()
            pltpu.make_async_copy(ones_v, dacc.at[didx.at[isl]],
                                  sems).wait()

        fire_idx(0)
        fire_idx(1)
        wait_idx(0)
        fire_gather(0)

        def body(i, carry):
            @pl.when(i >= 1)
            def _():
                wait_scat(i - 1)

            @pl.when(i + 2 < NBLK)
            def _():
                fire_idx(i + 2)

            @pl.when(i + 1 < NBLK)
            def _():
                wait_idx(i + 1)
                fire_gather(i + 1)

            wait_gather(i)
            fire_scat(i)
            return carry

        lax.fori_loop(0, NBLK, body, 0)
        wait_scat(NBLK - 1)
        plsc.subcore_barrier()

        for kk in range(ROUNDS):
            cid = s + NS * kk

            @pl.when(cid < NRCHUNK)
            def _():
                r0 = cid * RCHUNK
                pltpu.sync_copy(acc.at[pl.ds(r0, RCHUNK)], rows.at[0])
                pltpu.sync_copy(rows.at[0], acc_out.at[c, pl.ds(r0, RCHUNK)])
                pltpu.sync_copy(dacc.at[pl.ds(r0, RCHUNK)], dstage)

                @pl.when(c == 0)
                def _():
                    pltpu.sync_copy(dstage, deg0_out.at[pl.ds(r0, RCHUNK)])

                @pl.when(c == 1)
                def _():
                    pltpu.sync_copy(dstage, deg1_out.at[pl.ds(r0, RCHUNK)])

    return k(ndata, src, dst, zrows, zdeg, ones_blk)


ROW_BLK = 400


def _tc_finish_body(nd_ref, p0_ref, p1_ref, d0_ref, d1_ref, wt_ref, b_ref,
                    o_ref):
    deg = d0_ref[...] + d1_ref[...]
    agg = (p0_ref[...] + p1_ref[...]) / jnp.maximum(deg, 1.0)
    rst = nd_ref[...] + agg
    o_ref[...] = (
        jnp.dot(rst, wt_ref[...], preferred_element_type=jnp.float32)
        + b_ref[...]
    )


def _tc_finish(ndata, p0, p1, d0, d1, wt, b2):
    grid = (N // ROW_BLK,)
    return pl.pallas_call(
        _tc_finish_body,
        grid=grid,
        in_specs=[
            pl.BlockSpec((ROW_BLK, D), lambda i: (i, 0)),
            pl.BlockSpec((ROW_BLK, D), lambda i: (i, 0)),
            pl.BlockSpec((ROW_BLK, D), lambda i: (i, 0)),
            pl.BlockSpec((ROW_BLK, 1), lambda i: (i, 0)),
            pl.BlockSpec((ROW_BLK, 1), lambda i: (i, 0)),
            pl.BlockSpec((D, D), lambda i: (0, 0)),
            pl.BlockSpec((1, D), lambda i: (0, 0)),
        ],
        out_specs=pl.BlockSpec((ROW_BLK, D), lambda i: (i, 0)),
        out_shape=jax.ShapeDtypeStruct((N, D), jnp.float32),
    )(ndata, p0, p1, d0, d1, wt, b2)


@jax.jit
def kernel(ndata, edge_index, W, b):
    src = edge_index[0]
    dst = edge_index[1]
    zrows = jnp.zeros((RCHUNK, D), jnp.float32)
    zdeg = jnp.zeros((RCHUNK,), jnp.float32)
    ones_blk = jnp.ones((BLK,), jnp.float32)

    acc, deg0, deg1 = _sc_aggregate(ndata, src, dst, zrows, zdeg, ones_blk)

    wt = W.T
    b2 = b.reshape(1, D)
    return _tc_finish(ndata, acc[0], acc[1], deg0.reshape(N, 1),
                      deg1.reshape(N, 1), wt, b2)


# speedup vs baseline: 13.4272x; 1.1394x over previous
import functools

import jax
import jax.numpy as jnp
from jax import lax
from jax.experimental import pallas as pl
from jax.experimental.pallas import tpu as pltpu
from jax.experimental.pallas import tpu_sc as plsc

N = 10000
E = 320000
D = 128

NC = 2
NS = 16
NW = NC * NS

EDGES_PER_WORKER = E // NW
BLK = 80
NBLK = EDGES_PER_WORKER // BLK
RCHUNK = 80
NRCHUNK = N // RCHUNK
ROUNDS = (NRCHUNK + NS - 1) // NS
RI = 6
RR = 4


def _sc_aggregate(ndata, src, dst, zrows, zdeg, ones_blk):
    mesh = plsc.VectorSubcoreMesh(core_axis_name="c", subcore_axis_name="s")

    @functools.partial(
        pl.kernel,
        mesh=mesh,
        out_type=(
            jax.ShapeDtypeStruct((NC, N, D), jnp.float32),
            jax.ShapeDtypeStruct((N,), jnp.float32),
            jax.ShapeDtypeStruct((N,), jnp.float32),
        ),
        scratch_types=[
            pltpu.VMEM((RI, BLK), jnp.int32),
            pltpu.VMEM((RI, BLK), jnp.int32),
            pltpu.VMEM((RR, BLK, D), jnp.float32),
            pltpu.VMEM((RCHUNK,), jnp.float32),
            pltpu.VMEM((BLK,), jnp.float32),
            pltpu.VMEM_SHARED((N, D), jnp.float32),
            pltpu.VMEM_SHARED((N,), jnp.float32),
            pltpu.SemaphoreType.DMA((RI,)),
            pltpu.SemaphoreType.DMA((RR,)),
            pltpu.SemaphoreType.DMA((2,)),
        ],
    )
    def k(ndata_hbm, src_hbm, dst_hbm, zrows_hbm, zdeg_hbm, ones_hbm,
          acc_out, deg0_out, deg1_out, sidx, didx, rows, dstage,
          ones_v, acc, dacc, semi, semg, sems):
        c = lax.axis_index("c")
        s = lax.axis_index("s")

        pltpu.sync_copy(zrows_hbm, rows.at[0])
        pltpu.sync_copy(zdeg_hbm, dstage)
        pltpu.sync_copy(ones_hbm, ones_v)
        for kk in range(ROUNDS):
            cid = s + NS * kk

            @pl.when(cid < NRCHUNK)
            def _():
                r0 = cid * RCHUNK
                pltpu.sync_copy(rows.at[0], acc.at[pl.ds(r0, RCHUNK)])
                pltpu.sync_copy(dstage, dacc.at[pl.ds(r0, RCHUNK)])

        plsc.subcore_barrier()

        chunk_base = (c * NS + s) * EDGES_PER_WORKER

        def fire_idx(blk):
            isl = lax.rem(blk, RI)
            base = chunk_base + blk * BLK
            pltpu.async_copy(src_hbm.at[pl.ds(base, BLK)], sidx.at[isl],
                             semi.at[isl])
            pltpu.async_copy(dst_hbm.at[pl.ds(base, BLK)], didx.at[isl],
                             semi.at[isl])

        def wait_idx(blk):
            isl = lax.rem(blk, RI)
            base = chunk_base + blk * BLK
            pltpu.make_async_copy(src_hbm.at[pl.ds(base, BLK)],
                                  sidx.at[isl], semi.at[isl]).wait()
            pltpu.make_async_copy(dst_hbm.at[pl.ds(base, BLK)],
                                  didx.at[isl], semi.at[isl]).wait()

        def fire_gather(blk):
            isl = lax.rem(blk, RI)
            rsl = lax.rem(blk, RR)
            pltpu.async_copy(ndata_hbm.at[sidx.at[isl]], rows.at[rsl],
                             semg.at[rsl])

        def wait_gather(blk):
            isl = lax.rem(blk, RI)
            rsl = lax.rem(blk, RR)
            pltpu.make_async_copy(ndata_hbm.at[sidx.at[isl]], rows.at[rsl],
                                  semg.at[rsl]).wait()

        def fire_scat(blk):
            isl = lax.rem(blk, RI)
            rsl = lax.rem(blk, RR)
            ssl = lax.rem(blk, 2)
            pltpu.async_copy(rows.at[rsl], acc.at[didx.at[isl]],
                             sems.at[ssl], add=True)
            pltpu.async_copy(ones_v, dacc.at[didx.at[isl]], sems.at[ssl],
                             add=True)

        def wait_scat(blk):
            isl = lax.rem(blk, RI)
            rsl = lax.rem(blk, RR)
            ssl = lax.rem(blk, 2)
            pltpu.make_async_copy(rows.at[rsl], acc.at[didx.at[isl]],
                                  sems.at[ssl]).wait()
            pltpu.make_async_copy(ones_v, dacc.at[didx.at[isl]],
                                  sems.at[ssl]).wait()

        fire_idx(0)
        fire_idx(1)
        fire_idx(2)
        fire_idx(3)
        wait_idx(0)
        fire_gather(0)
        wait_idx(1)
        fire_gather(1)

        def body(i, carry):
            @pl.when(i >= 2)
            def _():
                wait_scat(i - 2)

            @pl.when(i + 4 < NBLK)
            def _():
                fire_idx(i + 4)

            @pl.when(i + 2 < NBLK)
            def _():
                wait_idx(i + 2)
                fire_gather(i + 2)

            wait_gather(i)
            fire_scat(i)
            return carry

        lax.fori_loop(0, NBLK, body, 0)
        wait_scat(NBLK - 2)
        wait_scat(NBLK - 1)
        plsc.subcore_barrier()

        for kk in range(ROUNDS):
            cid = s + NS * kk

            @pl.when(cid < NRCHUNK)
            def _():
                r0 = cid * RCHUNK
                pltpu.sync_copy(acc.at[pl.ds(r0, RCHUNK)], rows.at[0])
                pltpu.sync_copy(rows.at[0], acc_out.at[c, pl.ds(r0, RCHUNK)])
                pltpu.sync_copy(dacc.at[pl.ds(r0, RCHUNK)], dstage)

                @pl.when(c == 0)
                def _():
                    pltpu.sync_copy(dstage, deg0_out.at[pl.ds(r0, RCHUNK)])

                @pl.when(c == 1)
                def _():
                    pltpu.sync_copy(dstage, deg1_out.at[pl.ds(r0, RCHUNK)])

    return k(ndata, src, dst, zrows, zdeg, ones_blk)


ROW_BLK = 400


def _tc_finish_body(nd_ref, p_ref, d0_ref, d1_ref, wt_ref, b_ref, o_ref):
    deg = d0_ref[...] + d1_ref[...]
    agg = (p_ref[0] + p_ref[1]) / jnp.maximum(deg, 1.0)
    rst = nd_ref[...] + agg
    o_ref[...] = (
        jnp.dot(rst, wt_ref[...], preferred_element_type=jnp.float32)
        + b_ref[...]
    )


def _tc_finish(ndata, p, d0, d1, wt, b2):
    grid = (N // ROW_BLK,)
    return pl.pallas_call(
        _tc_finish_body,
        grid=grid,
        in_specs=[
            pl.BlockSpec((ROW_BLK, D), lambda i: (i, 0)),
            pl.BlockSpec((NC, ROW_BLK, D), lambda i: (0, i, 0)),
            pl.BlockSpec((ROW_BLK, 1), lambda i: (i, 0)),
            pl.BlockSpec((ROW_BLK, 1), lambda i: (i, 0)),
            pl.BlockSpec((D, D), lambda i: (0, 0)),
            pl.BlockSpec((1, D), lambda i: (0, 0)),
        ],
        out_specs=pl.BlockSpec((ROW_BLK, D), lambda i: (i, 0)),
        out_shape=jax.ShapeDtypeStruct((N, D), jnp.float32),
    )(ndata, p, d0, d1, wt, b2)


@jax.jit
def kernel(ndata, edge_index, W, b):
    src = edge_index[0]
    dst = edge_index[1]
    zrows = jnp.zeros((RCHUNK, D), jnp.float32)
    zdeg = jnp.zeros((RCHUNK,), jnp.float32)
    ones_blk = jnp.ones((BLK,), jnp.float32)

    acc, deg0, deg1 = _sc_aggregate(ndata, src, dst, zrows, zdeg, ones_blk)

    wt = W.T
    b2 = b.reshape(1, D)
    return _tc_finish(ndata, acc, deg0.reshape(N, 1), deg1.reshape(N, 1),
                      wt, b2)


# speedup vs baseline: 14.9874x; 1.1162x over previous
import functools

import jax
import jax.numpy as jnp
from jax import lax
from jax.experimental import pallas as pl
from jax.experimental.pallas import tpu as pltpu
from jax.experimental.pallas import tpu_sc as plsc

N = 10000
E = 320000
D = 128

NC = 2
NS = 16
NW = NC * NS

EDGES_PER_WORKER = E // NW
BLK = 80
NBLK = EDGES_PER_WORKER // BLK
RCHUNK = 80
NRCHUNK = N // RCHUNK
ROUNDS = (NRCHUNK + NS - 1) // NS
RI = 6
RR = 4


def _sc_aggregate(ndata, eidx_flat, zrows, zdeg, ones_blk):
    mesh = plsc.VectorSubcoreMesh(core_axis_name="c", subcore_axis_name="s")

    @functools.partial(
        pl.kernel,
        mesh=mesh,
        out_type=(
            jax.ShapeDtypeStruct((NC, N, D), jnp.float32),
            jax.ShapeDtypeStruct((N,), jnp.float32),
            jax.ShapeDtypeStruct((N,), jnp.float32),
        ),
        scratch_types=[
            pltpu.VMEM((RI, BLK), jnp.int32),
            pltpu.VMEM((RI, BLK), jnp.int32),
            pltpu.VMEM((RR, BLK, D), jnp.float32),
            pltpu.VMEM((RCHUNK,), jnp.float32),
            pltpu.VMEM((BLK,), jnp.float32),
            pltpu.VMEM_SHARED((N, D), jnp.float32),
            pltpu.VMEM_SHARED((N,), jnp.float32),
            pltpu.SemaphoreType.DMA((RI,)),
            pltpu.SemaphoreType.DMA((RR,)),
            pltpu.SemaphoreType.DMA((2,)),
        ],
    )
    def k(ndata_hbm, eidx_hbm, zrows_hbm, zdeg_hbm, ones_hbm,
          acc_out, deg0_out, deg1_out, sidx, didx, rows, dstage,
          ones_v, acc, dacc, semi, semg, sems):
        c = lax.axis_index("c")
        s = lax.axis_index("s")

        pltpu.sync_copy(zrows_hbm, rows.at[0])
        pltpu.sync_copy(zdeg_hbm, dstage)
        pltpu.sync_copy(ones_hbm, ones_v)
        for kk in range(ROUNDS):
            cid = s + NS * kk

            @pl.when(cid < NRCHUNK)
            def _():
                r0 = cid * RCHUNK
                pltpu.sync_copy(rows.at[0], acc.at[pl.ds(r0, RCHUNK)])
                pltpu.sync_copy(dstage, dacc.at[pl.ds(r0, RCHUNK)])

        plsc.subcore_barrier()

        chunk_base = (c * NS + s) * EDGES_PER_WORKER

        def fire_idx(blk):
            isl = lax.rem(blk, RI)
            base = chunk_base + blk * BLK
            pltpu.async_copy(eidx_hbm.at[pl.ds(base, BLK)], sidx.at[isl],
                             semi.at[isl])
            pltpu.async_copy(eidx_hbm.at[pl.ds(E + base, BLK)],
                             didx.at[isl], semi.at[isl])

        def wait_idx(blk):
            isl = lax.rem(blk, RI)
            base = chunk_base + blk * BLK
            pltpu.make_async_copy(eidx_hbm.at[pl.ds(base, BLK)],
                                  sidx.at[isl], semi.at[isl]).wait()
            pltpu.make_async_copy(eidx_hbm.at[pl.ds(E + base, BLK)],
                                  didx.at[isl], semi.at[isl]).wait()

        def fire_gather(blk):
            isl = lax.rem(blk, RI)
            rsl = lax.rem(blk, RR)
            pltpu.async_copy(ndata_hbm.at[sidx.at[isl]], rows.at[rsl],
                             semg.at[rsl])

        def wait_gather(blk):
            isl = lax.rem(blk, RI)
            rsl = lax.rem(blk, RR)
            pltpu.make_async_copy(ndata_hbm.at[sidx.at[isl]], rows.at[rsl],
                                  semg.at[rsl]).wait()

        def fire_scat(blk):
            isl = lax.rem(blk, RI)
            rsl = lax.rem(blk, RR)
            ssl = lax.rem(blk, 2)
            pltpu.async_copy(rows.at[rsl], acc.at[didx.at[isl]],
                             sems.at[ssl], add=True)
            pltpu.async_copy(ones_v, dacc.at[didx.at[isl]], sems.at[ssl],
                             add=True)

        def wait_scat(blk):
            isl = lax.rem(blk, RI)
            rsl = lax.rem(blk, RR)
            ssl = lax.rem(blk, 2)
            pltpu.make_async_copy(rows.at[rsl], acc.at[didx.at[isl]],
                                  sems.at[ssl]).wait()
            pltpu.make_async_copy(ones_v, dacc.at[didx.at[isl]],
                                  sems.at[ssl]).wait()

        fire_idx(0)
        fire_idx(1)
        fire_idx(2)
        fire_idx(3)
        wait_idx(0)
        fire_gather(0)
        wait_idx(1)
        fire_gather(1)

        def body(i, carry):
            @pl.when(i >= 2)
            def _():
                wait_scat(i - 2)

            @pl.when(i + 4 < NBLK)
            def _():
                fire_idx(i + 4)

            @pl.when(i + 2 < NBLK)
            def _():
                wait_idx(i + 2)
                fire_gather(i + 2)

            wait_gather(i)
            fire_scat(i)
            return carry

        lax.fori_loop(0, NBLK, body, 0)
        wait_scat(NBLK - 2)
        wait_scat(NBLK - 1)
        plsc.subcore_barrier()

        for kk in range(ROUNDS):
            cid = s + NS * kk

            @pl.when(cid < NRCHUNK)
            def _():
                r0 = cid * RCHUNK
                pltpu.sync_copy(acc.at[pl.ds(r0, RCHUNK)], rows.at[0])
                pltpu.sync_copy(rows.at[0], acc_out.at[c, pl.ds(r0, RCHUNK)])
                pltpu.sync_copy(dacc.at[pl.ds(r0, RCHUNK)], dstage)

                @pl.when(c == 0)
                def _():
                    pltpu.sync_copy(dstage, deg0_out.at[pl.ds(r0, RCHUNK)])

                @pl.when(c == 1)
                def _():
                    pltpu.sync_copy(dstage, deg1_out.at[pl.ds(r0, RCHUNK)])

    return k(ndata, eidx_flat, zrows, zdeg, ones_blk)


ROW_BLK = 400


def _tc_finish_body(nd_ref, p_ref, d0_ref, d1_ref, wt_ref, b_ref, o_ref):
    deg = d0_ref[0] + d1_ref[0]
    dcol = jnp.transpose(deg)
    agg = (p_ref[0] + p_ref[1]) / jnp.maximum(dcol, 1.0)
    rst = nd_ref[...] + agg
    o_ref[...] = (
        jnp.dot(rst, wt_ref[...], preferred_element_type=jnp.float32)
        + b_ref[...]
    )


def _tc_finish(ndata, p, d0, d1, wt, b2):
    grid = (N // ROW_BLK,)
    return pl.pallas_call(
        _tc_finish_body,
        grid=grid,
        in_specs=[
            pl.BlockSpec((ROW_BLK, D), lambda i: (i, 0)),
            pl.BlockSpec((NC, ROW_BLK, D), lambda i: (0, i, 0)),
            pl.BlockSpec((1, 1, ROW_BLK), lambda i: (i, 0, 0)),
            pl.BlockSpec((1, 1, ROW_BLK), lambda i: (i, 0, 0)),
            pl.BlockSpec((D, D), lambda i: (0, 0)),
            pl.BlockSpec((1, D), lambda i: (0, 0)),
        ],
        out_specs=pl.BlockSpec((ROW_BLK, D), lambda i: (i, 0)),
        out_shape=jax.ShapeDtypeStruct((N, D), jnp.float32),
    )(ndata, p, d0, d1, wt, b2)


@jax.jit
def kernel(ndata, edge_index, W, b):
    eidx_flat = edge_index.reshape(2 * E)
    zrows = jnp.zeros((RCHUNK, D), jnp.float32)
    zdeg = jnp.zeros((RCHUNK,), jnp.float32)
    ones_blk = jnp.ones((BLK,), jnp.float32)

    acc, deg0, deg1 = _sc_aggregate(ndata, eidx_flat, zrows, zdeg, ones_blk)

    wt = W.T
    b2 = b.reshape(1, D)
    return _tc_finish(ndata, acc, deg0.reshape(N // ROW_BLK, 1, ROW_BLK),
                      deg1.reshape(N // ROW_BLK, 1, ROW_BLK), wt, b2)


# speedup vs baseline: 16.1277x; 1.0761x over previous
import functools

import jax
import jax.numpy as jnp
from jax import lax
from jax.experimental import pallas as pl
from jax.experimental.pallas import tpu as pltpu
from jax.experimental.pallas import tpu_sc as plsc

N = 10000
E = 320000
D = 128

NC = 2
NS = 16
NW = NC * NS

EDGES_PER_WORKER = E // NW
BLK = 80
NBLK = EDGES_PER_WORKER // BLK
RCHUNK = 80
NRCHUNK = N // RCHUNK
ROUNDS = (NRCHUNK + NS - 1) // NS
RI = 6
RR = 4


def _sc_aggregate(ndata, eidx_flat, zrows, zdeg, ones_blk):
    mesh = plsc.VectorSubcoreMesh(core_axis_name="c", subcore_axis_name="s")

    @functools.partial(
        pl.kernel,
        mesh=mesh,
        out_type=(
            jax.ShapeDtypeStruct((NC, N, D), jnp.float32),
            jax.ShapeDtypeStruct((N,), jnp.float32),
            jax.ShapeDtypeStruct((N,), jnp.float32),
        ),
        scratch_types=[
            pltpu.VMEM((RI, BLK), jnp.int32),
            pltpu.VMEM((RI, BLK), jnp.int32),
            pltpu.VMEM((RR, BLK, D), jnp.float32),
            pltpu.VMEM((RCHUNK,), jnp.float32),
            pltpu.VMEM((BLK,), jnp.float32),
            pltpu.VMEM_SHARED((N, D), jnp.float32),
            pltpu.VMEM_SHARED((N,), jnp.float32),
            pltpu.SemaphoreType.DMA((RI,)),
            pltpu.SemaphoreType.DMA((RR,)),
            pltpu.SemaphoreType.DMA((2,)),
        ],
    )
    def k(ndata_hbm, eidx_hbm, zrows_hbm, zdeg_hbm, ones_hbm,
          acc_out, deg0_out, deg1_out, sidx, didx, rows, dstage,
          ones_v, acc, dacc, semi, semg, sems):
        c = lax.axis_index("c")
        s = lax.axis_index("s")

        pltpu.sync_copy(zrows_hbm, rows.at[0])
        pltpu.sync_copy(zdeg_hbm, dstage)
        pltpu.sync_copy(ones_hbm, ones_v)
        for kk in range(ROUNDS):
            cid = s + NS * kk

            @pl.when(cid < NRCHUNK)
            def _():
                r0 = cid * RCHUNK
                pltpu.sync_copy(rows.at[0], acc.at[pl.ds(r0, RCHUNK)])
                pltpu.sync_copy(dstage, dacc.at[pl.ds(r0, RCHUNK)])

        plsc.subcore_barrier()

        chunk_base = (c * NS + s) * EDGES_PER_WORKER

        def fire_idx(blk):
            isl = lax.rem(blk, RI)
            base = chunk_base + blk * BLK
            pltpu.async_copy(eidx_hbm.at[pl.ds(base, BLK)], sidx.at[isl],
                             semi.at[isl])
            pltpu.async_copy(eidx_hbm.at[pl.ds(E + base, BLK)],
                             didx.at[isl], semi.at[isl])

        def wait_idx(blk):
            isl = lax.rem(blk, RI)
            base = chunk_base + blk * BLK
            pltpu.make_async_copy(eidx_hbm.at[pl.ds(base, BLK)],
                                  sidx.at[isl], semi.at[isl]).wait()
            pltpu.make_async_copy(eidx_hbm.at[pl.ds(E + base, BLK)],
                                  didx.at[isl], semi.at[isl]).wait()

        def fire_gather(blk):
            isl = lax.rem(blk, RI)
            rsl = lax.rem(blk, RR)
            pltpu.async_copy(ndata_hbm.at[sidx.at[isl]], rows.at[rsl],
                             semg.at[rsl])

        def wait_gather(blk):
            isl = lax.rem(blk, RI)
            rsl = lax.rem(blk, RR)
            pltpu.make_async_copy(ndata_hbm.at[sidx.at[isl]], rows.at[rsl],
                                  semg.at[rsl]).wait()

        def fire_scat(blk):
            isl = lax.rem(blk, RI)
            rsl = lax.rem(blk, RR)
            ssl = lax.rem(blk, 2)
            pltpu.async_copy(rows.at[rsl], acc.at[didx.at[isl]],
                             sems.at[ssl], add=True)
            pltpu.async_copy(ones_v, dacc.at[didx.at[isl]], sems.at[ssl],
                             add=True)

        def wait_scat(blk):
            isl = lax.rem(blk, RI)
            rsl = lax.rem(blk, RR)
            ssl = lax.rem(blk, 2)
            pltpu.make_async_copy(rows.at[rsl], acc.at[didx.at[isl]],
                                  sems.at[ssl]).wait()
            pltpu.make_async_copy(ones_v, dacc.at[didx.at[isl]],
                                  sems.at[ssl]).wait()

        fire_idx(0)
        fire_idx(1)
        fire_idx(2)
        fire_idx(3)
        wait_idx(0)
        fire_gather(0)
        wait_idx(1)
        fire_gather(1)

        def body(i, carry):
            @pl.when(i >= 2)
            def _():
                wait_scat(i - 2)

            @pl.when(i + 4 < NBLK)
            def _():
                fire_idx(i + 4)

            @pl.when(i + 2 < NBLK)
            def _():
                wait_idx(i + 2)
                fire_gather(i + 2)

            wait_gather(i)
            fire_scat(i)
            return carry

        lax.fori_loop(0, NBLK, body, 0)
        wait_scat(NBLK - 2)
        wait_scat(NBLK - 1)
        plsc.subcore_barrier()

        for kk in range(ROUNDS):
            cid = s + NS * kk

            @pl.when(cid < NRCHUNK)
            def _():
                r0 = cid * RCHUNK
                pltpu.sync_copy(acc.at[pl.ds(r0, RCHUNK)],
                                acc_out.at[c, pl.ds(r0, RCHUNK)])
                pltpu.sync_copy(dacc.at[pl.ds(r0, RCHUNK)], dstage)

                @pl.when(c == 0)
                def _():
                    pltpu.sync_copy(dstage, deg0_out.at[pl.ds(r0, RCHUNK)])

                @pl.when(c == 1)
                def _():
                    pltpu.sync_copy(dstage, deg1_out.at[pl.ds(r0, RCHUNK)])

    return k(ndata, eidx_flat, zrows, zdeg, ones_blk)


ROW_BLK = 1000


def _tc_finish_body(nd_ref, p_ref, d0_ref, d1_ref, wt_ref, b_ref, o_ref):
    deg = d0_ref[0] + d1_ref[0]
    dcol = jnp.transpose(deg)
    agg = (p_ref[0] + p_ref[1]) / jnp.maximum(dcol, 1.0)
    rst = nd_ref[...] + agg
    o_ref[...] = (
        jnp.dot(rst, wt_ref[...], preferred_element_type=jnp.float32)
        + b_ref[...]
    )


def _tc_finish(ndata, p, d0, d1, wt, b2):
    grid = (N // ROW_BLK,)
    return pl.pallas_call(
        _tc_finish_body,
        grid=grid,
        in_specs=[
            pl.BlockSpec((ROW_BLK, D), lambda i: (i, 0)),
            pl.BlockSpec((NC, ROW_BLK, D), lambda i: (0, i, 0)),
            pl.BlockSpec((1, 1, ROW_BLK), lambda i: (i, 0, 0)),
            pl.BlockSpec((1, 1, ROW_BLK), lambda i: (i, 0, 0)),
            pl.BlockSpec((D, D), lambda i: (0, 0)),
            pl.BlockSpec((1, D), lambda i: (0, 0)),
        ],
        out_specs=pl.BlockSpec((ROW_BLK, D), lambda i: (i, 0)),
        out_shape=jax.ShapeDtypeStruct((N, D), jnp.float32),
    )(ndata, p, d0, d1, wt, b2)


@jax.jit
def kernel(ndata, edge_index, W, b):
    eidx_flat = edge_index.reshape(2 * E)
    zrows = jnp.zeros((RCHUNK, D), jnp.float32)
    zdeg = jnp.zeros((RCHUNK,), jnp.float32)
    ones_blk = jnp.ones((BLK,), jnp.float32)

    acc, deg0, deg1 = _sc_aggregate(ndata, eidx_flat, zrows, zdeg, ones_blk)

    wt = W.T
    b2 = b.reshape(1, D)
    return _tc_finish(ndata, acc, deg0.reshape(N // ROW_BLK, 1, ROW_BLK),
                      deg1.reshape(N // ROW_BLK, 1, ROW_BLK), wt, b2)
